# Initial kernel scaffold; baseline (speedup 1.0000x reference)
import jax
import jax.numpy as jnp
from jax.experimental import pallas as pl


def kernel(h, x, edge_index, edge_dist, W1e, b1e, W2e, b2e, W1n, b1n, W2n, b2n, W1c, b1c, W2c):
    raise NotImplementedError("write your pallas kernel here")


# speedup vs baseline: 2.0272x; 2.0272x over previous
import functools

import jax
import jax.numpy as jnp
from jax import lax
from jax.experimental import pallas as pl
from jax.experimental.pallas import tpu as pltpu
from jax.experimental.pallas import tpu_sc as plsc

N, E, D, H, ED = 10000, 320000, 128, 128, 16
ROW = 144
NC, NS = 2, 16
NW = NC * NS
EW = E // NW
C = 80
NCHUNK = EW // C
ROWS_PER_TILE = N // NS

_mesh = plsc.VectorSubcoreMesh(core_axis_name="c", subcore_axis_name="s")
_sc_params = pltpu.CompilerParams(use_tc_tiling_on_sc=False)


@functools.partial(
    pl.kernel,
    out_type=(
        jax.ShapeDtypeStruct((E, ROW), jnp.float32),
        jax.ShapeDtypeStruct((E, ROW), jnp.float32),
    ),
    mesh=_mesh,
    compiler_params=_sc_params,
    scratch_types=[
        pltpu.VMEM((C,), jnp.int32),
        pltpu.VMEM((C,), jnp.int32),
        pltpu.VMEM((C, ROW), jnp.float32),
        pltpu.VMEM((C, ROW), jnp.float32),
        pltpu.SemaphoreType.DMA,
        pltpu.SemaphoreType.DMA,
    ],
)
def _gather_k(hx_hbm, src_hbm, dst_hbm, gs_hbm, gd_hbm,
              idx_s, idx_d, buf_s, buf_d, sem_s, sem_d):
    wid = lax.axis_index("s") * NC + lax.axis_index("c")
    base = wid * EW

    def body(i, carry):
        off = base + i * C
        pltpu.sync_copy(src_hbm.at[pl.ds(off, C)], idx_s)
        pltpu.sync_copy(dst_hbm.at[pl.ds(off, C)], idx_d)
        cp_s = pltpu.async_copy(hx_hbm.at[idx_s], buf_s, sem_s)
        cp_d = pltpu.async_copy(hx_hbm.at[idx_d], buf_d, sem_d)
        cp_s.wait()
        cp_d.wait()
        pltpu.sync_copy(buf_s, gs_hbm.at[pl.ds(off, C)])
        pltpu.sync_copy(buf_d, gd_hbm.at[pl.ds(off, C)])
        return carry

    lax.fori_loop(0, NCHUNK, body, 0)


EB = 512
GRID = E // EB


def _mlp_body(gs, gd, dist, w1e, b1e, w2e, b2e,
              w1na, w1nb, w1ne, b1n, w2n, b2n,
              w1ca, w1cb, w1ce, b1c, w2c,
              m_out, cu_out):
    d = dist[...]
    a1 = d * w1e[...] + b1e[...]
    a1 = a1 * jax.nn.sigmoid(a1)
    attr = jnp.dot(a1, w2e[...], preferred_element_type=jnp.float32) + b2e[...]
    hs = gs[:, :D]
    hd = gd[:, :D]
    pre_n = (jnp.dot(hs, w1na[...], preferred_element_type=jnp.float32)
             + jnp.dot(hd, w1nb[...], preferred_element_type=jnp.float32)
             + jnp.dot(attr, w1ne[...], preferred_element_type=jnp.float32)
             + b1n[...])
    hid_n = pre_n * jax.nn.sigmoid(pre_n)
    m_out[...] = jnp.dot(hid_n, w2n[...], preferred_element_type=jnp.float32) + b2n[...]
    pre_c = (jnp.dot(hs, w1ca[...], preferred_element_type=jnp.float32)
             + jnp.dot(hd, w1cb[...], preferred_element_type=jnp.float32)
             + jnp.dot(attr, w1ce[...], preferred_element_type=jnp.float32)
             + b1c[...])
    hid_c = pre_c * jax.nn.sigmoid(pre_c)
    w = jnp.dot(hid_c, w2c[...], preferred_element_type=jnp.float32)
    cu_out[...] = w * (gs[:, D:ROW] - gd[:, D:ROW])


def _full(shape):
    return pl.BlockSpec(shape, lambda i: (0, 0))


_mlp_call = pl.pallas_call(
    _mlp_body,
    grid=(GRID,),
    in_specs=[
        pl.BlockSpec((EB, ROW), lambda i: (i, 0)),
        pl.BlockSpec((EB, ROW), lambda i: (i, 0)),
        pl.BlockSpec((EB, 1), lambda i: (i, 0)),
        _full((1, ED)), _full((1, ED)), _full((ED, ED)), _full((1, ED)),
        _full((D, H)), _full((D, H)), _full((ED, H)), _full((1, H)),
        _full((H, D)), _full((1, D)),
        _full((D, H)), _full((D, H)), _full((ED, H)), _full((1, H)),
        _full((H, 1)),
    ],
    out_specs=[
        pl.BlockSpec((EB, D), lambda i: (i, 0)),
        pl.BlockSpec((EB, 16), lambda i: (i, 0)),
    ],
    out_shape=[
        jax.ShapeDtypeStruct((E, D), jnp.float32),
        jax.ShapeDtypeStruct((E, 16), jnp.float32),
    ],
)


@functools.partial(
    pl.kernel,
    out_type=(
        jax.ShapeDtypeStruct((NC, N, D), jnp.float32),
        jax.ShapeDtypeStruct((NC, N, 16), jnp.float32),
    ),
    mesh=_mesh,
    compiler_params=_sc_params,
    scratch_types=[
        pltpu.VMEM((C,), jnp.int32),
        pltpu.VMEM((C, D), jnp.float32),
        pltpu.VMEM((C, 16), jnp.float32),
        pltpu.VMEM_SHARED((N, D), jnp.float32),
        pltpu.VMEM_SHARED((N, 16), jnp.float32),
    ],
)
def _scatter_k(m_hbm, cu_hbm, dst_hbm, z128_hbm, z16_hbm, hp_hbm, xp_hbm,
               idx_d, m_buf, cu_buf, h_acc, x_acc):
    cid = lax.axis_index("c")
    sid = lax.axis_index("s")
    wid = sid * NC + cid
    r0 = sid * ROWS_PER_TILE
    pltpu.sync_copy(z128_hbm.at[pl.ds(r0, ROWS_PER_TILE)],
                    h_acc.at[pl.ds(r0, ROWS_PER_TILE)])
    pltpu.sync_copy(z16_hbm.at[pl.ds(r0, ROWS_PER_TILE)],
                    x_acc.at[pl.ds(r0, ROWS_PER_TILE)])
    plsc.subcore_barrier()
    base = wid * EW

    def body(i, carry):
        off = base + i * C
        pltpu.sync_copy(dst_hbm.at[pl.ds(off, C)], idx_d)
        pltpu.sync_copy(m_hbm.at[pl.ds(off, C)], m_buf)
        pltpu.sync_copy(cu_hbm.at[pl.ds(off, C)], cu_buf)
        pltpu.sync_copy(m_buf, h_acc.at[idx_d], add=True)
        pltpu.sync_copy(cu_buf, x_acc.at[idx_d], add=True)
        return carry

    lax.fori_loop(0, NCHUNK, body, 0)
    plsc.subcore_barrier()
    pltpu.sync_copy(h_acc.at[pl.ds(r0, ROWS_PER_TILE)],
                    hp_hbm.at[cid, pl.ds(r0, ROWS_PER_TILE)])
    pltpu.sync_copy(x_acc.at[pl.ds(r0, ROWS_PER_TILE)],
                    xp_hbm.at[cid, pl.ds(r0, ROWS_PER_TILE)])


def kernel(h, x, edge_index, edge_dist,
           W1e, b1e, W2e, b2e, W1n, b1n, W2n, b2n, W1c, b1c, W2c):
    src = edge_index[0]
    dst = edge_index[1]
    hx = jnp.concatenate(
        [h, x, jnp.zeros((N, ROW - D - 3), jnp.float32)], axis=1)
    gs, gd = _gather_k(hx, src, dst)
    m, cu = _mlp_call(
        gs, gd, edge_dist.reshape(E, 1),
        W1e, b1e.reshape(1, ED), W2e, b2e.reshape(1, ED),
        W1n[:D], W1n[D:2 * D], W1n[2 * D:], b1n.reshape(1, H),
        W2n, b2n.reshape(1, D),
        W1c[:D], W1c[D:2 * D], W1c[2 * D:], b1c.reshape(1, H),
        W2c,
    )
    z128 = jnp.zeros((N, D), jnp.float32)
    z16 = jnp.zeros((N, 16), jnp.float32)
    hp, xp = _scatter_k(m, cu, dst, z128, z16)
    h_out = h + hp[0] + hp[1]
    x_out = x + xp[0, :, :3] + xp[1, :, :3]
    return (h_out, x_out)
